# baseline (speedup 1.0000x reference)
import jax
import jax.numpy as jnp
import numpy as np
from jax import lax
from jax.experimental import pallas as pl
from jax.experimental.pallas import tpu as pltpu, tpu_sc as plsc

_VOCAB = 100000
_D = 1024
_BLOCK = 2048
_BATCH = 4

_NC = 2
_NS = 16
_NW = _NC * _NS
_L = 16

_P_PER_W = _BLOCK // _NW
_C = 8
_NPC = _P_PER_W // _C
_R = _BATCH * _C
_NBUF = 3


def _positional_encoding(length, d_model):
    pos = np.arange(length, dtype=np.float32)[:, np.newaxis]
    i = np.arange(d_model, dtype=np.float32)[np.newaxis, :]
    angle_rates = 1.0 / np.power(
        10000.0, (2.0 * np.floor(i / 2.0)) / np.float32(d_model))
    angle_rads = pos * angle_rates
    angle_rads[:, 0::2] = np.sin(angle_rads[:, 0::2])
    angle_rads[:, 1::2] = np.cos(angle_rads[:, 1::2])
    return angle_rads


_PE = jnp.asarray(_positional_encoding(_BLOCK, _D), dtype=jnp.float32)


def _body(xp_hbm, pe_hbm, w_hbm, out_hbm, idx_v,
          pe0, pe1, pe2, rows0, rows1, rows2,
          gsem0, gsem1, gsem2, wsem0, wsem1, wsem2):
    pe_v = (pe0, pe1, pe2)
    rows = (rows0, rows1, rows2)
    gsem = (gsem0, gsem1, gsem2)
    wsem = (wsem0, wsem1, wsem2)
    wid = lax.axis_index("s") * _NC + lax.axis_index("c")
    pos0 = wid * _P_PER_W

    pltpu.sync_copy(xp_hbm.at[wid], idx_v)

    def issue_inputs(pc, slot):
        base = pos0 + pc * _C
        return [
            pltpu.async_copy(
                pe_hbm.at[pl.ds(base, _C), :], pe_v[slot], gsem[slot]),
            pltpu.async_copy(
                w_hbm.at[idx_v.at[pc]], rows[slot], gsem[slot]),
        ]

    def issue_writes(pc, slot):
        base = pos0 + pc * _C
        return [pltpu.async_copy(
            rows[slot].at[pl.ds(b * _C, _C), :],
            out_hbm.at[b, pl.ds(base, _C), :], wsem[slot])
            for b in range(_BATCH)]

    in_descs = {pc: issue_inputs(pc, pc % _NBUF) for pc in range(2)}
    out_descs = {}
    for pc in range(_NPC):
        slot = pc % _NBUF
        for d in in_descs.pop(pc):
            d.wait()
        nxt = pc + 2
        if nxt < _NPC:
            prev = nxt - _NBUF
            if prev >= 0:
                for d in out_descs.pop(prev):
                    d.wait()
            in_descs[nxt] = issue_inputs(nxt, nxt % _NBUF)

        @pl.loop(0, _C)
        def _row(rr):
            @pl.loop(0, _D // _L, unroll=8)
            def _col(cc):
                sl = pl.ds(cc * _L, _L)
                pe = pe_v[slot][rr, sl]
                for b in range(_BATCH):
                    plsc.addupdate(rows[slot].at[b * _C + rr, sl], pe)

        out_descs[pc] = issue_writes(pc, slot)

    for pc in sorted(out_descs):
        for d in out_descs[pc]:
            d.wait()


@jax.jit
def _run(x, pe, w):
    xp = jnp.transpose(
        x.reshape(_BATCH, _NW, _NPC, _C), (1, 2, 0, 3)
    ).reshape(_NW, _NPC, _R)
    mesh = plsc.VectorSubcoreMesh(core_axis_name="c", subcore_axis_name="s")
    f = pl.kernel(
        _body,
        out_type=jax.ShapeDtypeStruct((_BATCH, _BLOCK, _D), jnp.float32),
        mesh=mesh,
        scratch_types=(
            [pltpu.VMEM((_NPC, _R), jnp.int32)]
            + [pltpu.VMEM((_C, _D), jnp.float32) for _ in range(_NBUF)]
            + [pltpu.VMEM((_R, _D), jnp.float32) for _ in range(_NBUF)]
            + [pltpu.SemaphoreType.DMA for _ in range(2 * _NBUF)]
        ),
    )
    return f(xp, pe, w)


def kernel(x, W):
    return _run(x.astype(jnp.int32), _PE, W)


# speedup vs baseline: 1.0105x; 1.0105x over previous
import jax
import jax.numpy as jnp
import numpy as np
from jax import lax
from jax.experimental import pallas as pl
from jax.experimental.pallas import tpu as pltpu, tpu_sc as plsc

_VOCAB = 100000
_D = 1024
_BLOCK = 2048
_BATCH = 4

_NC = 2
_NS = 16
_NW = _NC * _NS
_L = 16

_P_PER_W = _BLOCK // _NW
_C = 8
_NPC = _P_PER_W // _C
_R = _BATCH * _C
_NBUF = 3


def _positional_encoding(length, d_model):
    pos = np.arange(length, dtype=np.float32)[:, np.newaxis]
    i = np.arange(d_model, dtype=np.float32)[np.newaxis, :]
    angle_rates = 1.0 / np.power(
        10000.0, (2.0 * np.floor(i / 2.0)) / np.float32(d_model))
    angle_rads = pos * angle_rates
    angle_rads[:, 0::2] = np.sin(angle_rads[:, 0::2])
    angle_rads[:, 1::2] = np.cos(angle_rads[:, 1::2])
    return angle_rads


_PE = jnp.asarray(_positional_encoding(_BLOCK, _D), dtype=jnp.float32)


def _body(xp_hbm, pe_hbm, w_hbm, out_hbm, idx_v,
          pe0, pe1, pe2, rows0, rows1, rows2,
          gsem0, gsem1, gsem2, wsem0, wsem1, wsem2, isem):
    pe_v = (pe0, pe1, pe2)
    rows = (rows0, rows1, rows2)
    gsem = (gsem0, gsem1, gsem2)
    wsem = (wsem0, wsem1, wsem2)
    wid = lax.axis_index("s") * _NC + lax.axis_index("c")
    pos0 = wid * _P_PER_W

    idx_desc = pltpu.async_copy(xp_hbm.at[wid], idx_v, isem)

    def issue_pe(pc, slot):
        base = pos0 + pc * _C
        return pltpu.async_copy(
            pe_hbm.at[pl.ds(base, _C), :], pe_v[slot], gsem[slot])

    def issue_gather(pc, slot):
        return pltpu.async_copy(
            w_hbm.at[idx_v.at[pc]], rows[slot], gsem[slot])

    def issue_inputs(pc, slot):
        return [issue_pe(pc, slot), issue_gather(pc, slot)]

    def issue_writes(pc, slot):
        base = pos0 + pc * _C
        return [pltpu.async_copy(
            rows[slot].at[pl.ds(b * _C, _C), :],
            out_hbm.at[b, pl.ds(base, _C), :], wsem[slot])
            for b in range(_BATCH)]

    pe_descs = [issue_pe(pc, pc % _NBUF) for pc in range(2)]
    idx_desc.wait()
    in_descs = {pc: [pe_descs[pc], issue_gather(pc, pc % _NBUF)]
                for pc in range(2)}
    out_descs = {}
    for pc in range(_NPC):
        slot = pc % _NBUF
        for d in in_descs.pop(pc):
            d.wait()
        nxt = pc + 2
        if nxt < _NPC:
            prev = nxt - _NBUF
            if prev >= 0:
                for d in out_descs.pop(prev):
                    d.wait()
            in_descs[nxt] = issue_inputs(nxt, nxt % _NBUF)

        @pl.loop(0, _C)
        def _row(rr):
            @plsc.parallel_loop(0, _D // _L, unroll=8)
            def _col(cc):
                sl = pl.ds(cc * _L, _L)
                pe = pe_v[slot][rr, sl]
                for b in range(_BATCH):
                    plsc.addupdate(rows[slot].at[b * _C + rr, sl], pe)

        out_descs[pc] = issue_writes(pc, slot)

    for pc in sorted(out_descs):
        for d in out_descs[pc]:
            d.wait()


@jax.jit
def _run(x, pe, w):
    xp = jnp.transpose(
        x.reshape(_BATCH, _NW, _NPC, _C), (1, 2, 0, 3)
    ).reshape(_NW, _NPC, _R)
    mesh = plsc.VectorSubcoreMesh(core_axis_name="c", subcore_axis_name="s")
    f = pl.kernel(
        _body,
        out_type=jax.ShapeDtypeStruct((_BATCH, _BLOCK, _D), jnp.float32),
        mesh=mesh,
        scratch_types=(
            [pltpu.VMEM((_NPC, _R), jnp.int32)]
            + [pltpu.VMEM((_C, _D), jnp.float32) for _ in range(_NBUF)]
            + [pltpu.VMEM((_R, _D), jnp.float32) for _ in range(_NBUF)]
            + [pltpu.SemaphoreType.DMA for _ in range(2 * _NBUF + 1)]
        ),
    )
    return f(xp, pe, w)


def kernel(x, W):
    return _run(x.astype(jnp.int32), _PE, W)
